# baseline (speedup 1.0000x reference)
import functools

import jax
import jax.numpy as jnp
from jax import lax
from jax.experimental import pallas as pl
from jax.experimental.pallas import tpu as pltpu
from jax.experimental.pallas import tpu_sc as plsc

N = 10000
E = 160000
D = 128
H = 256

_BF = jnp.bfloat16

_NC = 2
_NS = 16
_NW = _NC * _NS

_CH = 200
_CHS = 40
_EPW = E // _NW
_EPT = E // _NC // _NS
_NPT = 624
_NTAIL = N - _NS * _NPT


def _pack2(x):
    rb = x.astype(_BF).astype(jnp.float32)
    bu = jax.lax.bitcast_convert_type(rb, jnp.uint32)
    pk = (bu[:, 0:D] >> 16) | (bu[:, D:2 * D] & jnp.uint32(0xFFFF0000))
    return jax.lax.bitcast_convert_type(pk, jnp.int32)


def _unpk2(p):
    pu = jax.lax.bitcast_convert_type(p, jnp.uint32)
    lo = jax.lax.bitcast_convert_type(pu << 16, jnp.float32)
    hi = jax.lax.bitcast_convert_type(pu & jnp.uint32(0xFFFF0000), jnp.float32)
    return lo, hi


def _sc_gather_body(tabs_hbm, tabr_hbm, s_hbm, r_hbm, out_hbm,
                    s_all, r_all, rs0, rr0, rs1, rr1, sg0, sg1, sw0, sw1):
    wid = lax.axis_index("s") * _NC + lax.axis_index("c")
    base = wid * _EPW
    n_ch = _EPW // _CH

    pltpu.sync_copy(s_hbm.at[pl.ds(base, _EPW)], s_all)
    pltpu.sync_copy(r_hbm.at[pl.ds(base, _EPW)], r_all)

    def start_gather(j, rs, rr, sg):
        off = j * _CH
        pltpu.async_copy(tabs_hbm.at[s_all.at[pl.ds(off, _CH)]], rs, sg)
        pltpu.async_copy(tabr_hbm.at[r_all.at[pl.ds(off, _CH)]], rr, sg)

    def wait_gather(rs, rr, sg):
        pltpu.make_async_copy(tabs_hbm.at[pl.ds(0, _CH)], rs, sg).wait()
        pltpu.make_async_copy(tabr_hbm.at[pl.ds(0, _CH)], rr, sg).wait()

    def start_wb(j, rs, rr, sw):
        off = base + j * _CH
        pltpu.async_copy(rs, out_hbm.at[0, pl.ds(off, _CH)], sw)
        pltpu.async_copy(rr, out_hbm.at[1, pl.ds(off, _CH)], sw)

    def wait_wb(rs, rr, sw):
        pltpu.make_async_copy(rs, out_hbm.at[0, pl.ds(base, _CH)], sw).wait()
        pltpu.make_async_copy(rr, out_hbm.at[1, pl.ds(base, _CH)], sw).wait()

    def _iter(j, rs, rr, sg, sw, ors, orr, osg, osw):
        @pl.when(j + 1 < n_ch)
        def _():
            @pl.when(j >= 1)
            def _():
                wait_wb(ors, orr, osw)
            start_gather(j + 1, ors, orr, osg)
        wait_gather(rs, rr, sg)
        start_wb(j, rs, rr, sw)

    start_gather(0, rs0, rr0, sg0)

    def body(j, _):
        @pl.when(j % 2 == 0)
        def _():
            _iter(j, rs0, rr0, sg0, sw0, rs1, rr1, sg1, sw1)

        @pl.when(j % 2 == 1)
        def _():
            _iter(j, rs1, rr1, sg1, sw1, rs0, rr0, sg0, sw0)

        return 0

    lax.fori_loop(0, n_ch, body, 0)
    wait_wb(rs0, rr0, sw0)
    wait_wb(rs1, rr1, sw1)


def _sc_gather(tabs, tabr, senders, receivers):
    kfn = functools.partial(
        pl.kernel,
        out_type=jax.ShapeDtypeStruct((2, E, D), jnp.int32),
        mesh=plsc.VectorSubcoreMesh(core_axis_name="c", subcore_axis_name="s"),
        scratch_types=[
            pltpu.VMEM((_EPW,), jnp.int32),
            pltpu.VMEM((_EPW,), jnp.int32),
            pltpu.VMEM((_CH, D), jnp.int32),
            pltpu.VMEM((_CH, D), jnp.int32),
            pltpu.VMEM((_CH, D), jnp.int32),
            pltpu.VMEM((_CH, D), jnp.int32),
            pltpu.SemaphoreType.DMA,
            pltpu.SemaphoreType.DMA,
            pltpu.SemaphoreType.DMA,
            pltpu.SemaphoreType.DMA,
        ],
    )(_sc_gather_body)
    return kfn(tabs, tabr, senders, receivers)


def _sc_scatter_body(e_hbm, r_hbm, z_hbm, out_hbm,
                     rv0, ev0, rv1, ev1, acc_sh, sl0, sl1, ss0, ss1):
    cid = lax.axis_index("c")
    sid = lax.axis_index("s")
    nbase = sid * _NPT
    base = cid * (E // _NC) + sid * _EPT
    n_ch = _EPT // _CHS

    def start_load(j, rv, ev, sl):
        off = base + j * _CHS
        pltpu.async_copy(r_hbm.at[pl.ds(off, _CHS)], rv, sl)
        pltpu.async_copy(e_hbm.at[pl.ds(off, _CHS)], ev, sl)

    def wait_load(rv, ev, sl):
        pltpu.make_async_copy(r_hbm.at[pl.ds(0, _CHS)], rv, sl).wait()
        pltpu.make_async_copy(e_hbm.at[pl.ds(0, _CHS)], ev, sl).wait()

    def start_scat(rv, ev, ss):
        pltpu.async_copy(ev, acc_sh.at[rv], ss, add=True)

    def wait_scat(rv, ev, ss):
        pltpu.make_async_copy(ev, acc_sh.at[rv], ss).wait()

    start_load(0, rv0, ev0, sl0)

    pltpu.sync_copy(z_hbm.at[pl.ds(nbase, _NPT)], acc_sh.at[pl.ds(nbase, _NPT)])

    @pl.when(sid == _NS - 1)
    def _():
        pltpu.sync_copy(z_hbm.at[pl.ds(_NS * _NPT, _NTAIL)],
                        acc_sh.at[pl.ds(_NS * _NPT, _NTAIL)])

    plsc.subcore_barrier()

    def _iter(j, rv, ev, sl, ss, orv, oev, osl, oss):
        @pl.when(j + 1 < n_ch)
        def _():
            @pl.when(j >= 1)
            def _():
                wait_scat(orv, oev, oss)
            start_load(j + 1, orv, oev, osl)
        wait_load(rv, ev, sl)
        start_scat(rv, ev, ss)

    def body(j, _):
        @pl.when(j % 2 == 0)
        def _():
            _iter(j, rv0, ev0, sl0, ss0, rv1, ev1, sl1, ss1)

        @pl.when(j % 2 == 1)
        def _():
            _iter(j, rv1, ev1, sl1, ss1, rv0, ev0, sl0, ss0)

        return 0

    lax.fori_loop(0, n_ch, body, 0)
    wait_scat(rv0, ev0, ss0)
    wait_scat(rv1, ev1, ss1)
    plsc.subcore_barrier()
    pltpu.sync_copy(acc_sh.at[pl.ds(nbase, _NPT)],
                    out_hbm.at[cid, pl.ds(nbase, _NPT)])

    @pl.when(sid == _NS - 1)
    def _():
        pltpu.sync_copy(acc_sh.at[pl.ds(_NS * _NPT, _NTAIL)],
                        out_hbm.at[cid, pl.ds(_NS * _NPT, _NTAIL)])


def _sc_scatter(e_hat, receivers, zeros_nd):
    kfn = functools.partial(
        pl.kernel,
        out_type=jax.ShapeDtypeStruct((2, N, D), jnp.float32),
        mesh=plsc.VectorSubcoreMesh(core_axis_name="c", subcore_axis_name="s"),
        scratch_types=[
            pltpu.VMEM((_CHS,), jnp.int32),
            pltpu.VMEM((_CHS, D), jnp.float32),
            pltpu.VMEM((_CHS,), jnp.int32),
            pltpu.VMEM((_CHS, D), jnp.float32),
            pltpu.VMEM_SHARED((N, D), jnp.float32),
            pltpu.SemaphoreType.DMA,
            pltpu.SemaphoreType.DMA,
            pltpu.SemaphoreType.DMA,
            pltpu.SemaphoreType.DMA,
        ],
    )(_sc_scatter_body)
    return kfn(e_hat, receivers, zeros_nd)


_BE = 2000
_BN = 2000


def _dot(a, b):
    return jax.lax.dot_general(a, b, (((1,), (0,)), ((), ())),
                               preferred_element_type=jnp.float32)


def _full(shape):
    return pl.BlockSpec(shape, lambda i: (0,) * len(shape))


def _prologue_body(obj_ref, ow1_ref, ob1_ref, ow2_ref, ob2_ref,
                   ea_w1a_ref, ea_b1_ref, wa_ref, wb_ref,
                   op_ref, qs_ref, qr_ref):
    o = obj_ref[...]
    h = jnp.maximum(_dot(o, ow1_ref[...]) + ob1_ref[...], 0.0)
    enc = _dot(h, ow2_ref[...]) + ob2_ref[...]
    op_ref[...] = _dot(enc, ea_w1a_ref[...]) + ea_b1_ref[...]
    ob = o.astype(_BF)
    qs_ref[...] = _pack2(_dot(ob, wa_ref[...]))
    qr_ref[...] = _pack2(_dot(ob, wb_ref[...]))


def _prologue(obj, oe_W1, oe_b1, oe_W2, oe_b2, ea_W1a, ea_b1, wAb, wBb):
    return pl.pallas_call(
        _prologue_body,
        grid=(N // _BN,),
        in_specs=[
            pl.BlockSpec((_BN, D), lambda i: (i, 0)),
            _full((D, H)),
            _full((1, H)),
            _full((H, D)),
            _full((1, D)),
            _full((D, H)),
            _full((1, H)),
            _full((D, H)),
            _full((D, H)),
        ],
        out_specs=[
            pl.BlockSpec((_BN, H), lambda i: (i, 0)),
            pl.BlockSpec((_BN, D), lambda i: (i, 0)),
            pl.BlockSpec((_BN, D), lambda i: (i, 0)),
        ],
        out_shape=[
            jax.ShapeDtypeStruct((N, H), jnp.float32),
            jax.ShapeDtypeStruct((N, D), jnp.int32),
            jax.ShapeDtypeStruct((N, D), jnp.int32),
        ],
    )(obj, oe_W1, oe_b1.reshape(1, H), oe_W2, oe_b2.reshape(1, D),
      ea_W1a, ea_b1.reshape(1, H), wAb, wBb)


def _pass0_body(g_ref, rel_ref, wc_ref, wa_ref, b1_ref, w2_ref, b2_ref,
                rp_ref, e1_ref):
    relc = _dot(rel_ref[...].astype(_BF), wc_ref[...]) + b1_ref[...]
    loS, hiS = _unpk2(g_ref[0])
    loR, hiR = _unpk2(g_ref[1])
    hL = jnp.maximum(relc[:, 0:D] + loS + loR, 0.0)
    hR = jnp.maximum(relc[:, D:2 * D] + hiS + hiR, 0.0)
    renc = (_dot(hL.astype(_BF), w2_ref[0:D]) +
            _dot(hR.astype(_BF), w2_ref[D:2 * D]) + b2_ref[...])
    rp = _dot(renc.astype(_BF), wa_ref[...]) + b1_ref[...]
    rp_ref[...] = _pack2(rp)
    e1_ref[...] = _dot(jnp.maximum(rp, 0.0).astype(_BF), w2_ref[...]) + b2_ref[...]


def _pass0(g0, rel, wCb, wAb, re_b1, re_W2b, re_b2):
    return pl.pallas_call(
        _pass0_body,
        grid=(E // _BE,),
        in_specs=[
            pl.BlockSpec((2, _BE, D), lambda i: (0, i, 0)),
            pl.BlockSpec((_BE, D), lambda i: (i, 0)),
            _full((D, H)),
            _full((D, H)),
            _full((1, H)),
            _full((H, D)),
            _full((1, D)),
        ],
        out_specs=[
            pl.BlockSpec((_BE, D), lambda i: (i, 0)),
            pl.BlockSpec((_BE, D), lambda i: (i, 0)),
        ],
        out_shape=[
            jax.ShapeDtypeStruct((E, D), jnp.int32),
            jax.ShapeDtypeStruct((E, D), jnp.float32),
        ],
    )(g0, rel, wCb, wAb, re_b1.reshape(1, H), re_W2b, re_b2.reshape(1, D))


def _edge_body(rp_ref, g_ref, w2_ref, b2_ref, out_ref):
    loS, hiS = _unpk2(g_ref[0])
    loR, hiR = _unpk2(g_ref[1])
    loP, hiP = _unpk2(rp_ref[...])
    hL = jnp.maximum(loP + loS + loR, 0.0)
    hR = jnp.maximum(hiP + hiS + hiR, 0.0)
    out_ref[...] = (_dot(hL.astype(_BF), w2_ref[0:D]) +
                    _dot(hR.astype(_BF), w2_ref[D:2 * D]) + b2_ref[...])


def _edge(rel_part, g, re_W2b, re_b2):
    return pl.pallas_call(
        _edge_body,
        grid=(E // _BE,),
        in_specs=[
            pl.BlockSpec((_BE, D), lambda i: (i, 0)),
            pl.BlockSpec((2, _BE, D), lambda i: (0, i, 0)),
            _full((H, D)),
            _full((1, D)),
        ],
        out_specs=pl.BlockSpec((_BE, D), lambda i: (i, 0)),
        out_shape=jax.ShapeDtypeStruct((E, D), jnp.float32),
    )(rel_part, g, re_W2b, re_b2.reshape(1, D))


def _node_body(op_ref, agg_ref, v_ref, w1_ref, w2_ref, b2_ref,
               wb_ref, wc_ref, v_out, ps_ref, pr_ref, *, with_v):
    agg = agg_ref[0] + agg_ref[1]
    h = op_ref[...] + _dot(agg, w1_ref[0:D])
    if with_v:
        h = h + _dot(v_ref[...], w1_ref[D:2 * D])
    h = jnp.maximum(h, 0.0)
    v = _dot(h, w2_ref[...]) + b2_ref[...]
    v_out[...] = v
    vb = v.astype(_BF)
    ps_ref[...] = _pack2(_dot(vb, wb_ref[...]))
    pr_ref[...] = _pack2(_dot(vb, wc_ref[...]))


def _node(obj_part, agg, v, ea_W1bc, ea_W2, ea_b2, wBb, wCb, with_v):
    return pl.pallas_call(
        functools.partial(_node_body, with_v=with_v),
        grid=(N // _BN,),
        in_specs=[
            pl.BlockSpec((_BN, H), lambda i: (i, 0)),
            pl.BlockSpec((2, _BN, D), lambda i: (0, i, 0)),
            pl.BlockSpec((_BN, D), lambda i: (i, 0)),
            _full((2 * D, H)),
            _full((H, D)),
            _full((1, D)),
            _full((D, H)),
            _full((D, H)),
        ],
        out_specs=[
            pl.BlockSpec((_BN, D), lambda i: (i, 0)),
            pl.BlockSpec((_BN, D), lambda i: (i, 0)),
            pl.BlockSpec((_BN, D), lambda i: (i, 0)),
        ],
        out_shape=[
            jax.ShapeDtypeStruct((N, D), jnp.float32),
            jax.ShapeDtypeStruct((N, D), jnp.int32),
            jax.ShapeDtypeStruct((N, D), jnp.int32),
        ],
    )(obj_part, agg, v, ea_W1bc, ea_W2, ea_b2.reshape(1, D), wBb, wCb)


def _node_final_body(op_ref, agg_ref, v_ref, w1_ref, w2_ref, b2_ref,
                     dw1_ref, db1_ref, dw2_ref, db2_ref, out_ref):
    agg = agg_ref[0] + agg_ref[1]
    h = op_ref[...] + _dot(agg, w1_ref[0:D]) + _dot(v_ref[...], w1_ref[D:2 * D])
    h = jnp.maximum(h, 0.0)
    v = _dot(h, w2_ref[...]) + b2_ref[...]
    h2 = jnp.maximum(_dot(v, dw1_ref[...]) + db1_ref[...], 0.0)
    out_ref[...] = _dot(h2, dw2_ref[...]) + db2_ref[...]


def _node_final(obj_part, agg, v, ea_W1bc, ea_W2, ea_b2,
                od_W1, od_b1, od_W2, od_b2):
    return pl.pallas_call(
        _node_final_body,
        grid=(N // _BN,),
        in_specs=[
            pl.BlockSpec((_BN, H), lambda i: (i, 0)),
            pl.BlockSpec((2, _BN, D), lambda i: (0, i, 0)),
            pl.BlockSpec((_BN, D), lambda i: (i, 0)),
            _full((2 * D, H)),
            _full((H, D)),
            _full((1, D)),
            _full((D, H)),
            _full((1, H)),
            _full((H, D)),
            _full((1, D)),
        ],
        out_specs=pl.BlockSpec((_BN, D), lambda i: (i, 0)),
        out_shape=jax.ShapeDtypeStruct((N, D), jnp.float32),
    )(obj_part, agg, v, ea_W1bc, ea_W2, ea_b2.reshape(1, D),
      od_W1, od_b1.reshape(1, H), od_W2, od_b2.reshape(1, D))


def kernel(objects, relations, senders, receivers,
           re_W1, re_b1, re_W2, re_b2,
           oe_W1, oe_b1, oe_W2, oe_b2,
           ea_W1, ea_b1, ea_W2, ea_b2,
           od_W1, od_b1, od_W2, od_b2):
    assert objects.shape == (1, N, D) and relations.shape == (1, E, D)
    obj = objects[0]
    rel = relations[0]

    wAb = re_W1[0:D].astype(_BF)
    wBb = re_W1[D:2 * D].astype(_BF)
    wCb = re_W1[2 * D:3 * D].astype(_BF)
    re_W2b = re_W2.astype(_BF)
    ea_W1bc = ea_W1[D:3 * D]

    obj_part, qs_p, qr_p = _prologue(obj, oe_W1, oe_b1, oe_W2, oe_b2,
                                     ea_W1[0:D], ea_b1, wAb, wBb)
    g0 = _sc_gather(qs_p, qr_p, senders, receivers)
    rel_part, e_hat = _pass0(g0, rel, wCb, wAb, re_b1, re_W2b, re_b2)

    zeros_nd = jnp.zeros((N, D), jnp.float32)

    agg = _sc_scatter(e_hat, receivers, zeros_nd)
    v, ps_p, pr_p = _node(obj_part, agg, zeros_nd, ea_W1bc, ea_W2, ea_b2,
                          wBb, wCb, with_v=False)

    g = _sc_gather(ps_p, pr_p, senders, receivers)
    e_hat = _edge(rel_part, g, re_W2b, re_b2)
    agg = _sc_scatter(e_hat, receivers, zeros_nd)
    v, ps_p, pr_p = _node(obj_part, agg, v, ea_W1bc, ea_W2, ea_b2,
                          wBb, wCb, with_v=True)

    g = _sc_gather(ps_p, pr_p, senders, receivers)
    e_hat = _edge(rel_part, g, re_W2b, re_b2)
    agg = _sc_scatter(e_hat, receivers, zeros_nd)
    out = _node_final(obj_part, agg, v, ea_W1bc, ea_W2, ea_b2,
                      od_W1, od_b1, od_W2, od_b2)
    return out[None]


# speedup vs baseline: 1.0750x; 1.0750x over previous
import functools

import jax
import jax.numpy as jnp
from jax import lax
from jax.experimental import pallas as pl
from jax.experimental.pallas import tpu as pltpu
from jax.experimental.pallas import tpu_sc as plsc

N = 10000
E = 160000
D = 128
H = 256

_BF = jnp.bfloat16

_NC = 2
_NS = 16
_NW = _NC * _NS

_CH = 200
_CHS = 40
_EPW = E // _NW
_EPT = E // _NC // _NS
_NPT = 624
_NTAIL = N - _NS * _NPT


def _pack2(x):
    rb = x.astype(_BF).astype(jnp.float32)
    bu = jax.lax.bitcast_convert_type(rb, jnp.uint32)
    pk = (bu[:, 0:D] >> 16) | (bu[:, D:2 * D] & jnp.uint32(0xFFFF0000))
    return jax.lax.bitcast_convert_type(pk, jnp.int32)


def _unpk2(p):
    pu = jax.lax.bitcast_convert_type(p, jnp.uint32)
    lo = jax.lax.bitcast_convert_type(pu << 16, jnp.float32)
    hi = jax.lax.bitcast_convert_type(pu & jnp.uint32(0xFFFF0000), jnp.float32)
    return lo, hi


def _sc_gather_body(tabs_hbm, tabr_hbm, s_hbm, r_hbm, out_hbm,
                    s_all, r_all, rs0, rr0, rs1, rr1, sg0, sg1, sw0, sw1):
    wid = lax.axis_index("s") * _NC + lax.axis_index("c")
    base = wid * _EPW
    n_ch = _EPW // _CH

    pltpu.sync_copy(s_hbm.at[pl.ds(base, _EPW)], s_all)
    pltpu.sync_copy(r_hbm.at[pl.ds(base, _EPW)], r_all)

    def start_gather(j, rs, rr, sg):
        off = j * _CH
        pltpu.async_copy(tabs_hbm.at[s_all.at[pl.ds(off, _CH)]], rs, sg)
        pltpu.async_copy(tabr_hbm.at[r_all.at[pl.ds(off, _CH)]], rr, sg)

    def wait_gather(rs, rr, sg):
        pltpu.make_async_copy(tabs_hbm.at[pl.ds(0, _CH)], rs, sg).wait()
        pltpu.make_async_copy(tabr_hbm.at[pl.ds(0, _CH)], rr, sg).wait()

    def start_wb(j, rs, rr, sw):
        off = base + j * _CH
        pltpu.async_copy(rs, out_hbm.at[0, pl.ds(off, _CH)], sw)
        pltpu.async_copy(rr, out_hbm.at[1, pl.ds(off, _CH)], sw)

    def wait_wb(rs, rr, sw):
        pltpu.make_async_copy(rs, out_hbm.at[0, pl.ds(base, _CH)], sw).wait()
        pltpu.make_async_copy(rr, out_hbm.at[1, pl.ds(base, _CH)], sw).wait()

    def _iter(j, rs, rr, sg, sw, ors, orr, osg, osw):
        @pl.when(j + 1 < n_ch)
        def _():
            @pl.when(j >= 1)
            def _():
                wait_wb(ors, orr, osw)
            start_gather(j + 1, ors, orr, osg)
        wait_gather(rs, rr, sg)
        start_wb(j, rs, rr, sw)

    start_gather(0, rs0, rr0, sg0)

    def body(j, _):
        @pl.when(j % 2 == 0)
        def _():
            _iter(j, rs0, rr0, sg0, sw0, rs1, rr1, sg1, sw1)

        @pl.when(j % 2 == 1)
        def _():
            _iter(j, rs1, rr1, sg1, sw1, rs0, rr0, sg0, sw0)

        return 0

    lax.fori_loop(0, n_ch, body, 0)
    wait_wb(rs0, rr0, sw0)
    wait_wb(rs1, rr1, sw1)


def _sc_gather(tabs, tabr, senders, receivers):
    kfn = functools.partial(
        pl.kernel,
        out_type=jax.ShapeDtypeStruct((2, E, D), jnp.int32),
        mesh=plsc.VectorSubcoreMesh(core_axis_name="c", subcore_axis_name="s"),
        scratch_types=[
            pltpu.VMEM((_EPW,), jnp.int32),
            pltpu.VMEM((_EPW,), jnp.int32),
            pltpu.VMEM((_CH, D), jnp.int32),
            pltpu.VMEM((_CH, D), jnp.int32),
            pltpu.VMEM((_CH, D), jnp.int32),
            pltpu.VMEM((_CH, D), jnp.int32),
            pltpu.SemaphoreType.DMA,
            pltpu.SemaphoreType.DMA,
            pltpu.SemaphoreType.DMA,
            pltpu.SemaphoreType.DMA,
        ],
    )(_sc_gather_body)
    return kfn(tabs, tabr, senders, receivers)


def _sc_scatter_body(e_hbm, r_hbm, z_hbm, out_hbm,
                     rv0, ev0, rv1, ev1, acc_sh, sl0, sl1, ss0, ss1):
    cid = lax.axis_index("c")
    sid = lax.axis_index("s")
    nbase = sid * _NPT
    base = cid * (E // _NC) + sid * _EPT
    n_ch = _EPT // _CHS

    def start_load(j, rv, ev, sl):
        off = base + j * _CHS
        pltpu.async_copy(r_hbm.at[pl.ds(off, _CHS)], rv, sl)
        pltpu.async_copy(e_hbm.at[pl.ds(off, _CHS)], ev, sl)

    def wait_load(rv, ev, sl):
        pltpu.make_async_copy(r_hbm.at[pl.ds(0, _CHS)], rv, sl).wait()
        pltpu.make_async_copy(e_hbm.at[pl.ds(0, _CHS)], ev, sl).wait()

    def start_scat(rv, ev, ss):
        pltpu.async_copy(ev, acc_sh.at[rv], ss, add=True)

    def wait_scat(rv, ev, ss):
        pltpu.make_async_copy(ev, acc_sh.at[rv], ss).wait()

    start_load(0, rv0, ev0, sl0)

    pltpu.sync_copy(z_hbm.at[pl.ds(nbase, _NPT)], acc_sh.at[pl.ds(nbase, _NPT)])

    @pl.when(sid == _NS - 1)
    def _():
        pltpu.sync_copy(z_hbm.at[pl.ds(_NS * _NPT, _NTAIL)],
                        acc_sh.at[pl.ds(_NS * _NPT, _NTAIL)])

    plsc.subcore_barrier()

    def _iter(j, rv, ev, sl, ss, orv, oev, osl, oss):
        @pl.when(j + 1 < n_ch)
        def _():
            @pl.when(j >= 1)
            def _():
                wait_scat(orv, oev, oss)
            start_load(j + 1, orv, oev, osl)
        wait_load(rv, ev, sl)
        start_scat(rv, ev, ss)

    def body(j, _):
        @pl.when(j % 2 == 0)
        def _():
            _iter(j, rv0, ev0, sl0, ss0, rv1, ev1, sl1, ss1)

        @pl.when(j % 2 == 1)
        def _():
            _iter(j, rv1, ev1, sl1, ss1, rv0, ev0, sl0, ss0)

        return 0

    lax.fori_loop(0, n_ch, body, 0)
    wait_scat(rv0, ev0, ss0)
    wait_scat(rv1, ev1, ss1)
    plsc.subcore_barrier()
    pltpu.sync_copy(acc_sh.at[pl.ds(nbase, _NPT)],
                    out_hbm.at[cid, pl.ds(nbase, _NPT)])

    @pl.when(sid == _NS - 1)
    def _():
        pltpu.sync_copy(acc_sh.at[pl.ds(_NS * _NPT, _NTAIL)],
                        out_hbm.at[cid, pl.ds(_NS * _NPT, _NTAIL)])


def _sc_scatter(e_hat, receivers, zeros_nd):
    kfn = functools.partial(
        pl.kernel,
        out_type=jax.ShapeDtypeStruct((2, N, D), jnp.float32),
        mesh=plsc.VectorSubcoreMesh(core_axis_name="c", subcore_axis_name="s"),
        scratch_types=[
            pltpu.VMEM((_CHS,), jnp.int32),
            pltpu.VMEM((_CHS, D), jnp.float32),
            pltpu.VMEM((_CHS,), jnp.int32),
            pltpu.VMEM((_CHS, D), jnp.float32),
            pltpu.VMEM_SHARED((N, D), jnp.float32),
            pltpu.SemaphoreType.DMA,
            pltpu.SemaphoreType.DMA,
            pltpu.SemaphoreType.DMA,
            pltpu.SemaphoreType.DMA,
        ],
    )(_sc_scatter_body)
    return kfn(e_hat, receivers, zeros_nd)


_BE = 4000
_BN = 2000


def _dot(a, b):
    return jax.lax.dot_general(a, b, (((1,), (0,)), ((), ())),
                               preferred_element_type=jnp.float32)


def _full(shape):
    return pl.BlockSpec(shape, lambda i: (0,) * len(shape))


def _prologue_body(obj_ref, ow1_ref, ob1_ref, ow2_ref, ob2_ref,
                   ea_w1a_ref, ea_b1_ref, wa_ref, wb_ref,
                   op_ref, qs_ref, qr_ref):
    o = obj_ref[...]
    h = jnp.maximum(_dot(o, ow1_ref[...]) + ob1_ref[...], 0.0)
    enc = _dot(h, ow2_ref[...]) + ob2_ref[...]
    op_ref[...] = _dot(enc, ea_w1a_ref[...]) + ea_b1_ref[...]
    ob = o.astype(_BF)
    qs_ref[...] = _pack2(_dot(ob, wa_ref[...]))
    qr_ref[...] = _pack2(_dot(ob, wb_ref[...]))


def _prologue(obj, oe_W1, oe_b1, oe_W2, oe_b2, ea_W1a, ea_b1, wAb, wBb):
    return pl.pallas_call(
        _prologue_body,
        grid=(N // _BN,),
        in_specs=[
            pl.BlockSpec((_BN, D), lambda i: (i, 0)),
            _full((D, H)),
            _full((1, H)),
            _full((H, D)),
            _full((1, D)),
            _full((D, H)),
            _full((1, H)),
            _full((D, H)),
            _full((D, H)),
        ],
        out_specs=[
            pl.BlockSpec((_BN, H), lambda i: (i, 0)),
            pl.BlockSpec((_BN, D), lambda i: (i, 0)),
            pl.BlockSpec((_BN, D), lambda i: (i, 0)),
        ],
        out_shape=[
            jax.ShapeDtypeStruct((N, H), jnp.float32),
            jax.ShapeDtypeStruct((N, D), jnp.int32),
            jax.ShapeDtypeStruct((N, D), jnp.int32),
        ],
    )(obj, oe_W1, oe_b1.reshape(1, H), oe_W2, oe_b2.reshape(1, D),
      ea_W1a, ea_b1.reshape(1, H), wAb, wBb)


def _pass0_body(g_ref, rel_ref, wc_ref, wa_ref, b1_ref, w2_ref, b2_ref,
                rp_ref, e1_ref):
    relc = _dot(rel_ref[...].astype(_BF), wc_ref[...]) + b1_ref[...]
    loS, hiS = _unpk2(g_ref[0])
    loR, hiR = _unpk2(g_ref[1])
    hL = jnp.maximum(relc[:, 0:D] + loS + loR, 0.0)
    hR = jnp.maximum(relc[:, D:2 * D] + hiS + hiR, 0.0)
    renc = (_dot(hL.astype(_BF), w2_ref[0:D]) +
            _dot(hR.astype(_BF), w2_ref[D:2 * D]) + b2_ref[...])
    rp = _dot(renc.astype(_BF), wa_ref[...]) + b1_ref[...]
    rp_ref[...] = _pack2(rp)
    e1_ref[...] = _dot(jnp.maximum(rp, 0.0).astype(_BF), w2_ref[...]) + b2_ref[...]


def _pass0(g0, rel, wCb, wAb, re_b1, re_W2b, re_b2):
    return pl.pallas_call(
        _pass0_body,
        grid=(E // _BE,),
        in_specs=[
            pl.BlockSpec((2, _BE, D), lambda i: (0, i, 0)),
            pl.BlockSpec((_BE, D), lambda i: (i, 0)),
            _full((D, H)),
            _full((D, H)),
            _full((1, H)),
            _full((H, D)),
            _full((1, D)),
        ],
        out_specs=[
            pl.BlockSpec((_BE, D), lambda i: (i, 0)),
            pl.BlockSpec((_BE, D), lambda i: (i, 0)),
        ],
        out_shape=[
            jax.ShapeDtypeStruct((E, D), jnp.int32),
            jax.ShapeDtypeStruct((E, D), jnp.float32),
        ],
    )(g0, rel, wCb, wAb, re_b1.reshape(1, H), re_W2b, re_b2.reshape(1, D))


def _edge_body(rp_ref, g_ref, w2_ref, b2_ref, out_ref):
    loS, hiS = _unpk2(g_ref[0])
    loR, hiR = _unpk2(g_ref[1])
    loP, hiP = _unpk2(rp_ref[...])
    hL = jnp.maximum(loP + loS + loR, 0.0)
    hR = jnp.maximum(hiP + hiS + hiR, 0.0)
    out_ref[...] = (_dot(hL.astype(_BF), w2_ref[0:D]) +
                    _dot(hR.astype(_BF), w2_ref[D:2 * D]) + b2_ref[...])


def _edge(rel_part, g, re_W2b, re_b2):
    return pl.pallas_call(
        _edge_body,
        grid=(E // _BE,),
        in_specs=[
            pl.BlockSpec((_BE, D), lambda i: (i, 0)),
            pl.BlockSpec((2, _BE, D), lambda i: (0, i, 0)),
            _full((H, D)),
            _full((1, D)),
        ],
        out_specs=pl.BlockSpec((_BE, D), lambda i: (i, 0)),
        out_shape=jax.ShapeDtypeStruct((E, D), jnp.float32),
    )(rel_part, g, re_W2b, re_b2.reshape(1, D))


def _node_body(op_ref, agg_ref, v_ref, w1_ref, w2_ref, b2_ref,
               wb_ref, wc_ref, v_out, ps_ref, pr_ref, *, with_v):
    agg = agg_ref[0] + agg_ref[1]
    h = op_ref[...] + _dot(agg, w1_ref[0:D])
    if with_v:
        h = h + _dot(v_ref[...], w1_ref[D:2 * D])
    h = jnp.maximum(h, 0.0)
    v = _dot(h, w2_ref[...]) + b2_ref[...]
    v_out[...] = v
    vb = v.astype(_BF)
    ps_ref[...] = _pack2(_dot(vb, wb_ref[...]))
    pr_ref[...] = _pack2(_dot(vb, wc_ref[...]))


def _node(obj_part, agg, v, ea_W1bc, ea_W2, ea_b2, wBb, wCb, with_v):
    return pl.pallas_call(
        functools.partial(_node_body, with_v=with_v),
        grid=(N // _BN,),
        in_specs=[
            pl.BlockSpec((_BN, H), lambda i: (i, 0)),
            pl.BlockSpec((2, _BN, D), lambda i: (0, i, 0)),
            pl.BlockSpec((_BN, D), lambda i: (i, 0)),
            _full((2 * D, H)),
            _full((H, D)),
            _full((1, D)),
            _full((D, H)),
            _full((D, H)),
        ],
        out_specs=[
            pl.BlockSpec((_BN, D), lambda i: (i, 0)),
            pl.BlockSpec((_BN, D), lambda i: (i, 0)),
            pl.BlockSpec((_BN, D), lambda i: (i, 0)),
        ],
        out_shape=[
            jax.ShapeDtypeStruct((N, D), jnp.float32),
            jax.ShapeDtypeStruct((N, D), jnp.int32),
            jax.ShapeDtypeStruct((N, D), jnp.int32),
        ],
    )(obj_part, agg, v, ea_W1bc, ea_W2, ea_b2.reshape(1, D), wBb, wCb)


def _node_final_body(op_ref, agg_ref, v_ref, w1_ref, w2_ref, b2_ref,
                     dw1_ref, db1_ref, dw2_ref, db2_ref, out_ref):
    agg = agg_ref[0] + agg_ref[1]
    h = op_ref[...] + _dot(agg, w1_ref[0:D]) + _dot(v_ref[...], w1_ref[D:2 * D])
    h = jnp.maximum(h, 0.0)
    v = _dot(h, w2_ref[...]) + b2_ref[...]
    h2 = jnp.maximum(_dot(v, dw1_ref[...]) + db1_ref[...], 0.0)
    out_ref[...] = _dot(h2, dw2_ref[...]) + db2_ref[...]


def _node_final(obj_part, agg, v, ea_W1bc, ea_W2, ea_b2,
                od_W1, od_b1, od_W2, od_b2):
    return pl.pallas_call(
        _node_final_body,
        grid=(N // _BN,),
        in_specs=[
            pl.BlockSpec((_BN, H), lambda i: (i, 0)),
            pl.BlockSpec((2, _BN, D), lambda i: (0, i, 0)),
            pl.BlockSpec((_BN, D), lambda i: (i, 0)),
            _full((2 * D, H)),
            _full((H, D)),
            _full((1, D)),
            _full((D, H)),
            _full((1, H)),
            _full((H, D)),
            _full((1, D)),
        ],
        out_specs=pl.BlockSpec((_BN, D), lambda i: (i, 0)),
        out_shape=jax.ShapeDtypeStruct((N, D), jnp.float32),
    )(obj_part, agg, v, ea_W1bc, ea_W2, ea_b2.reshape(1, D),
      od_W1, od_b1.reshape(1, H), od_W2, od_b2.reshape(1, D))


def kernel(objects, relations, senders, receivers,
           re_W1, re_b1, re_W2, re_b2,
           oe_W1, oe_b1, oe_W2, oe_b2,
           ea_W1, ea_b1, ea_W2, ea_b2,
           od_W1, od_b1, od_W2, od_b2):
    assert objects.shape == (1, N, D) and relations.shape == (1, E, D)
    obj = objects[0]
    rel = relations[0]

    wAb = re_W1[0:D].astype(_BF)
    wBb = re_W1[D:2 * D].astype(_BF)
    wCb = re_W1[2 * D:3 * D].astype(_BF)
    re_W2b = re_W2.astype(_BF)
    ea_W1bc = ea_W1[D:3 * D]

    obj_part, qs_p, qr_p = _prologue(obj, oe_W1, oe_b1, oe_W2, oe_b2,
                                     ea_W1[0:D], ea_b1, wAb, wBb)
    g0 = _sc_gather(qs_p, qr_p, senders, receivers)
    rel_part, e_hat = _pass0(g0, rel, wCb, wAb, re_b1, re_W2b, re_b2)

    zeros_nd = jnp.zeros((N, D), jnp.float32)

    agg = _sc_scatter(e_hat, receivers, zeros_nd)
    v, ps_p, pr_p = _node(obj_part, agg, zeros_nd, ea_W1bc, ea_W2, ea_b2,
                          wBb, wCb, with_v=False)

    g = _sc_gather(ps_p, pr_p, senders, receivers)
    e_hat = _edge(rel_part, g, re_W2b, re_b2)
    agg = _sc_scatter(e_hat, receivers, zeros_nd)
    v, ps_p, pr_p = _node(obj_part, agg, v, ea_W1bc, ea_W2, ea_b2,
                          wBb, wCb, with_v=True)

    g = _sc_gather(ps_p, pr_p, senders, receivers)
    e_hat = _edge(rel_part, g, re_W2b, re_b2)
    agg = _sc_scatter(e_hat, receivers, zeros_nd)
    out = _node_final(obj_part, agg, v, ea_W1bc, ea_W2, ea_b2,
                      od_W1, od_b1, od_W2, od_b2)
    return out[None]


# speedup vs baseline: 1.0947x; 1.0184x over previous
import functools

import jax
import jax.numpy as jnp
from jax import lax
from jax.experimental import pallas as pl
from jax.experimental.pallas import tpu as pltpu
from jax.experimental.pallas import tpu_sc as plsc

N = 10000
E = 160000
D = 128
H = 256

_BF = jnp.bfloat16

_NC = 2
_NS = 16
_NW = _NC * _NS

_CH = 200
_CHS = 40
_EPW = E // _NW
_EPT = E // _NC // _NS
_NPT = 624
_NTAIL = N - _NS * _NPT


def _pack2(x):
    rb = x.astype(_BF).astype(jnp.float32)
    bu = jax.lax.bitcast_convert_type(rb, jnp.uint32)
    pk = (bu[:, 0:D] >> 16) | (bu[:, D:2 * D] & jnp.uint32(0xFFFF0000))
    return jax.lax.bitcast_convert_type(pk, jnp.int32)


def _unpk2(p):
    pu = jax.lax.bitcast_convert_type(p, jnp.uint32)
    lo = jax.lax.bitcast_convert_type(pu << 16, jnp.float32)
    hi = jax.lax.bitcast_convert_type(pu & jnp.uint32(0xFFFF0000), jnp.float32)
    return lo, hi


def _sc_gather_body(tabs_hbm, tabr_hbm, s_hbm, r_hbm, out_hbm,
                    s_all, r_all, rs0, rr0, rs1, rr1, sg0, sg1, sw0, sw1):
    wid = lax.axis_index("s") * _NC + lax.axis_index("c")
    base = wid * _EPW
    n_ch = _EPW // _CH

    pltpu.sync_copy(s_hbm.at[pl.ds(base, _EPW)], s_all)
    pltpu.sync_copy(r_hbm.at[pl.ds(base, _EPW)], r_all)

    def start_gather(j, rs, rr, sg):
        off = j * _CH
        pltpu.async_copy(tabs_hbm.at[s_all.at[pl.ds(off, _CH)]], rs, sg)
        pltpu.async_copy(tabr_hbm.at[r_all.at[pl.ds(off, _CH)]], rr, sg)

    def wait_gather(rs, rr, sg):
        pltpu.make_async_copy(tabs_hbm.at[pl.ds(0, _CH)], rs, sg).wait()
        pltpu.make_async_copy(tabr_hbm.at[pl.ds(0, _CH)], rr, sg).wait()

    def start_wb(j, rs, rr, sw):
        off = base + j * _CH
        pltpu.async_copy(rs, out_hbm.at[0, pl.ds(off, _CH)], sw)
        pltpu.async_copy(rr, out_hbm.at[1, pl.ds(off, _CH)], sw)

    def wait_wb(rs, rr, sw):
        pltpu.make_async_copy(rs, out_hbm.at[0, pl.ds(base, _CH)], sw).wait()
        pltpu.make_async_copy(rr, out_hbm.at[1, pl.ds(base, _CH)], sw).wait()

    def _iter(j, rs, rr, sg, sw, ors, orr, osg, osw):
        @pl.when(j + 1 < n_ch)
        def _():
            @pl.when(j >= 1)
            def _():
                wait_wb(ors, orr, osw)
            start_gather(j + 1, ors, orr, osg)
        wait_gather(rs, rr, sg)
        start_wb(j, rs, rr, sw)

    start_gather(0, rs0, rr0, sg0)

    def body(j, _):
        @pl.when(j % 2 == 0)
        def _():
            _iter(j, rs0, rr0, sg0, sw0, rs1, rr1, sg1, sw1)

        @pl.when(j % 2 == 1)
        def _():
            _iter(j, rs1, rr1, sg1, sw1, rs0, rr0, sg0, sw0)

        return 0

    lax.fori_loop(0, n_ch, body, 0)
    wait_wb(rs0, rr0, sw0)
    wait_wb(rs1, rr1, sw1)


def _sc_gather(tabs, tabr, senders, receivers):
    kfn = functools.partial(
        pl.kernel,
        out_type=jax.ShapeDtypeStruct((2, E, D), jnp.int32),
        mesh=plsc.VectorSubcoreMesh(core_axis_name="c", subcore_axis_name="s"),
        scratch_types=[
            pltpu.VMEM((_EPW,), jnp.int32),
            pltpu.VMEM((_EPW,), jnp.int32),
            pltpu.VMEM((_CH, D), jnp.int32),
            pltpu.VMEM((_CH, D), jnp.int32),
            pltpu.VMEM((_CH, D), jnp.int32),
            pltpu.VMEM((_CH, D), jnp.int32),
            pltpu.SemaphoreType.DMA,
            pltpu.SemaphoreType.DMA,
            pltpu.SemaphoreType.DMA,
            pltpu.SemaphoreType.DMA,
        ],
    )(_sc_gather_body)
    return kfn(tabs, tabr, senders, receivers)


def _sc_scatter_body(e_hbm, r_hbm, z_hbm, out_hbm,
                     rv0, ev0, rv1, ev1, acc_sh, sl0, sl1, ss0, ss1):
    cid = lax.axis_index("c")
    sid = lax.axis_index("s")
    nbase = sid * _NPT
    base = cid * (E // _NC) + sid * _EPT
    n_ch = _EPT // _CHS

    def start_load(j, rv, ev, sl):
        off = base + j * _CHS
        pltpu.async_copy(r_hbm.at[pl.ds(off, _CHS)], rv, sl)
        pltpu.async_copy(e_hbm.at[pl.ds(off, _CHS)], ev, sl)

    def wait_load(rv, ev, sl):
        pltpu.make_async_copy(r_hbm.at[pl.ds(0, _CHS)], rv, sl).wait()
        pltpu.make_async_copy(e_hbm.at[pl.ds(0, _CHS)], ev, sl).wait()

    def start_scat(rv, ev, ss):
        pltpu.async_copy(ev, acc_sh.at[rv], ss, add=True)

    def wait_scat(rv, ev, ss):
        pltpu.make_async_copy(ev, acc_sh.at[rv], ss).wait()

    start_load(0, rv0, ev0, sl0)

    pltpu.sync_copy(z_hbm.at[pl.ds(nbase, _NPT)], acc_sh.at[pl.ds(nbase, _NPT)])

    @pl.when(sid == _NS - 1)
    def _():
        pltpu.sync_copy(z_hbm.at[pl.ds(_NS * _NPT, _NTAIL)],
                        acc_sh.at[pl.ds(_NS * _NPT, _NTAIL)])

    plsc.subcore_barrier()

    def _iter(j, rv, ev, sl, ss, orv, oev, osl, oss):
        @pl.when(j + 1 < n_ch)
        def _():
            @pl.when(j >= 1)
            def _():
                wait_scat(orv, oev, oss)
            start_load(j + 1, orv, oev, osl)
        wait_load(rv, ev, sl)
        start_scat(rv, ev, ss)

    def body(j, _):
        @pl.when(j % 2 == 0)
        def _():
            _iter(j, rv0, ev0, sl0, ss0, rv1, ev1, sl1, ss1)

        @pl.when(j % 2 == 1)
        def _():
            _iter(j, rv1, ev1, sl1, ss1, rv0, ev0, sl0, ss0)

        return 0

    lax.fori_loop(0, n_ch, body, 0)
    wait_scat(rv0, ev0, ss0)
    wait_scat(rv1, ev1, ss1)
    plsc.subcore_barrier()
    pltpu.sync_copy(acc_sh.at[pl.ds(nbase, _NPT)],
                    out_hbm.at[cid, pl.ds(nbase, _NPT)])

    @pl.when(sid == _NS - 1)
    def _():
        pltpu.sync_copy(acc_sh.at[pl.ds(_NS * _NPT, _NTAIL)],
                        out_hbm.at[cid, pl.ds(_NS * _NPT, _NTAIL)])


def _sc_scatter(e_hat, receivers, zeros_nd):
    kfn = functools.partial(
        pl.kernel,
        out_type=jax.ShapeDtypeStruct((2, N, D), jnp.float32),
        mesh=plsc.VectorSubcoreMesh(core_axis_name="c", subcore_axis_name="s"),
        scratch_types=[
            pltpu.VMEM((_CHS,), jnp.int32),
            pltpu.VMEM((_CHS, D), jnp.float32),
            pltpu.VMEM((_CHS,), jnp.int32),
            pltpu.VMEM((_CHS, D), jnp.float32),
            pltpu.VMEM_SHARED((N, D), jnp.float32),
            pltpu.SemaphoreType.DMA,
            pltpu.SemaphoreType.DMA,
            pltpu.SemaphoreType.DMA,
            pltpu.SemaphoreType.DMA,
        ],
    )(_sc_scatter_body)
    return kfn(e_hat, receivers, zeros_nd)


_BE = 8000
_BN = 2000


def _dot(a, b):
    return jax.lax.dot_general(a, b, (((1,), (0,)), ((), ())),
                               preferred_element_type=jnp.float32)


def _full(shape):
    return pl.BlockSpec(shape, lambda i: (0,) * len(shape))


def _prologue_body(obj_ref, ow1_ref, ob1_ref, ow2_ref, ob2_ref,
                   ea_w1a_ref, ea_b1_ref, wa_ref, wb_ref,
                   op_ref, qs_ref, qr_ref):
    o = obj_ref[...]
    h = jnp.maximum(_dot(o, ow1_ref[...]) + ob1_ref[...], 0.0)
    enc = _dot(h, ow2_ref[...]) + ob2_ref[...]
    op_ref[...] = _dot(enc, ea_w1a_ref[...]) + ea_b1_ref[...]
    ob = o.astype(_BF)
    qs_ref[...] = _pack2(_dot(ob, wa_ref[...]))
    qr_ref[...] = _pack2(_dot(ob, wb_ref[...]))


def _prologue(obj, oe_W1, oe_b1, oe_W2, oe_b2, ea_W1a, ea_b1, wAb, wBb):
    return pl.pallas_call(
        _prologue_body,
        grid=(N // _BN,),
        in_specs=[
            pl.BlockSpec((_BN, D), lambda i: (i, 0)),
            _full((D, H)),
            _full((1, H)),
            _full((H, D)),
            _full((1, D)),
            _full((D, H)),
            _full((1, H)),
            _full((D, H)),
            _full((D, H)),
        ],
        out_specs=[
            pl.BlockSpec((_BN, H), lambda i: (i, 0)),
            pl.BlockSpec((_BN, D), lambda i: (i, 0)),
            pl.BlockSpec((_BN, D), lambda i: (i, 0)),
        ],
        out_shape=[
            jax.ShapeDtypeStruct((N, H), jnp.float32),
            jax.ShapeDtypeStruct((N, D), jnp.int32),
            jax.ShapeDtypeStruct((N, D), jnp.int32),
        ],
    )(obj, oe_W1, oe_b1.reshape(1, H), oe_W2, oe_b2.reshape(1, D),
      ea_W1a, ea_b1.reshape(1, H), wAb, wBb)


def _pass0_body(g_ref, rel_ref, wc_ref, wa_ref, b1_ref, w2_ref, b2_ref,
                rp_ref, e1_ref):
    relc = _dot(rel_ref[...].astype(_BF), wc_ref[...]) + b1_ref[...]
    loS, hiS = _unpk2(g_ref[0])
    loR, hiR = _unpk2(g_ref[1])
    hL = jnp.maximum(relc[:, 0:D] + loS + loR, 0.0)
    hR = jnp.maximum(relc[:, D:2 * D] + hiS + hiR, 0.0)
    renc = (_dot(hL.astype(_BF), w2_ref[0:D]) +
            _dot(hR.astype(_BF), w2_ref[D:2 * D]) + b2_ref[...])
    rp = _dot(renc.astype(_BF), wa_ref[...]) + b1_ref[...]
    rp_ref[...] = _pack2(rp)
    e1_ref[...] = _dot(jnp.maximum(rp, 0.0).astype(_BF), w2_ref[...]) + b2_ref[...]


def _pass0(g0, rel, wCb, wAb, re_b1, re_W2b, re_b2):
    return pl.pallas_call(
        _pass0_body,
        grid=(E // _BE,),
        in_specs=[
            pl.BlockSpec((2, _BE, D), lambda i: (0, i, 0)),
            pl.BlockSpec((_BE, D), lambda i: (i, 0)),
            _full((D, H)),
            _full((D, H)),
            _full((1, H)),
            _full((H, D)),
            _full((1, D)),
        ],
        out_specs=[
            pl.BlockSpec((_BE, D), lambda i: (i, 0)),
            pl.BlockSpec((_BE, D), lambda i: (i, 0)),
        ],
        out_shape=[
            jax.ShapeDtypeStruct((E, D), jnp.int32),
            jax.ShapeDtypeStruct((E, D), jnp.float32),
        ],
    )(g0, rel, wCb, wAb, re_b1.reshape(1, H), re_W2b, re_b2.reshape(1, D))


def _edge_body(rp_ref, g_ref, w2_ref, b2_ref, out_ref):
    loS, hiS = _unpk2(g_ref[0])
    loR, hiR = _unpk2(g_ref[1])
    loP, hiP = _unpk2(rp_ref[...])
    hL = jnp.maximum(loP + loS + loR, 0.0)
    hR = jnp.maximum(hiP + hiS + hiR, 0.0)
    out_ref[...] = (_dot(hL.astype(_BF), w2_ref[0:D]) +
                    _dot(hR.astype(_BF), w2_ref[D:2 * D]) + b2_ref[...])


def _edge(rel_part, g, re_W2b, re_b2):
    return pl.pallas_call(
        _edge_body,
        grid=(E // _BE,),
        in_specs=[
            pl.BlockSpec((_BE, D), lambda i: (i, 0)),
            pl.BlockSpec((2, _BE, D), lambda i: (0, i, 0)),
            _full((H, D)),
            _full((1, D)),
        ],
        out_specs=pl.BlockSpec((_BE, D), lambda i: (i, 0)),
        out_shape=jax.ShapeDtypeStruct((E, D), jnp.float32),
    )(rel_part, g, re_W2b, re_b2.reshape(1, D))


def _node_body(op_ref, agg_ref, v_ref, w1_ref, w2_ref, b2_ref,
               wb_ref, wc_ref, v_out, ps_ref, pr_ref, *, with_v):
    agg = agg_ref[0] + agg_ref[1]
    h = op_ref[...] + _dot(agg, w1_ref[0:D])
    if with_v:
        h = h + _dot(v_ref[...], w1_ref[D:2 * D])
    h = jnp.maximum(h, 0.0)
    v = _dot(h, w2_ref[...]) + b2_ref[...]
    v_out[...] = v
    vb = v.astype(_BF)
    ps_ref[...] = _pack2(_dot(vb, wb_ref[...]))
    pr_ref[...] = _pack2(_dot(vb, wc_ref[...]))


def _node(obj_part, agg, v, ea_W1bc, ea_W2, ea_b2, wBb, wCb, with_v):
    return pl.pallas_call(
        functools.partial(_node_body, with_v=with_v),
        grid=(N // _BN,),
        in_specs=[
            pl.BlockSpec((_BN, H), lambda i: (i, 0)),
            pl.BlockSpec((2, _BN, D), lambda i: (0, i, 0)),
            pl.BlockSpec((_BN, D), lambda i: (i, 0)),
            _full((2 * D, H)),
            _full((H, D)),
            _full((1, D)),
            _full((D, H)),
            _full((D, H)),
        ],
        out_specs=[
            pl.BlockSpec((_BN, D), lambda i: (i, 0)),
            pl.BlockSpec((_BN, D), lambda i: (i, 0)),
            pl.BlockSpec((_BN, D), lambda i: (i, 0)),
        ],
        out_shape=[
            jax.ShapeDtypeStruct((N, D), jnp.float32),
            jax.ShapeDtypeStruct((N, D), jnp.int32),
            jax.ShapeDtypeStruct((N, D), jnp.int32),
        ],
    )(obj_part, agg, v, ea_W1bc, ea_W2, ea_b2.reshape(1, D), wBb, wCb)


def _node_final_body(op_ref, agg_ref, v_ref, w1_ref, w2_ref, b2_ref,
                     dw1_ref, db1_ref, dw2_ref, db2_ref, out_ref):
    agg = agg_ref[0] + agg_ref[1]
    h = op_ref[...] + _dot(agg, w1_ref[0:D]) + _dot(v_ref[...], w1_ref[D:2 * D])
    h = jnp.maximum(h, 0.0)
    v = _dot(h, w2_ref[...]) + b2_ref[...]
    h2 = jnp.maximum(_dot(v, dw1_ref[...]) + db1_ref[...], 0.0)
    out_ref[...] = _dot(h2, dw2_ref[...]) + db2_ref[...]


def _node_final(obj_part, agg, v, ea_W1bc, ea_W2, ea_b2,
                od_W1, od_b1, od_W2, od_b2):
    return pl.pallas_call(
        _node_final_body,
        grid=(N // _BN,),
        in_specs=[
            pl.BlockSpec((_BN, H), lambda i: (i, 0)),
            pl.BlockSpec((2, _BN, D), lambda i: (0, i, 0)),
            pl.BlockSpec((_BN, D), lambda i: (i, 0)),
            _full((2 * D, H)),
            _full((H, D)),
            _full((1, D)),
            _full((D, H)),
            _full((1, H)),
            _full((H, D)),
            _full((1, D)),
        ],
        out_specs=pl.BlockSpec((_BN, D), lambda i: (i, 0)),
        out_shape=jax.ShapeDtypeStruct((N, D), jnp.float32),
    )(obj_part, agg, v, ea_W1bc, ea_W2, ea_b2.reshape(1, D),
      od_W1, od_b1.reshape(1, H), od_W2, od_b2.reshape(1, D))


def kernel(objects, relations, senders, receivers,
           re_W1, re_b1, re_W2, re_b2,
           oe_W1, oe_b1, oe_W2, oe_b2,
           ea_W1, ea_b1, ea_W2, ea_b2,
           od_W1, od_b1, od_W2, od_b2):
    assert objects.shape == (1, N, D) and relations.shape == (1, E, D)
    obj = objects[0]
    rel = relations[0]

    wAb = re_W1[0:D].astype(_BF)
    wBb = re_W1[D:2 * D].astype(_BF)
    wCb = re_W1[2 * D:3 * D].astype(_BF)
    re_W2b = re_W2.astype(_BF)
    ea_W1bc = ea_W1[D:3 * D]

    obj_part, qs_p, qr_p = _prologue(obj, oe_W1, oe_b1, oe_W2, oe_b2,
                                     ea_W1[0:D], ea_b1, wAb, wBb)
    g0 = _sc_gather(qs_p, qr_p, senders, receivers)
    rel_part, e_hat = _pass0(g0, rel, wCb, wAb, re_b1, re_W2b, re_b2)

    zeros_nd = jnp.zeros((N, D), jnp.float32)

    agg = _sc_scatter(e_hat, receivers, zeros_nd)
    v, ps_p, pr_p = _node(obj_part, agg, zeros_nd, ea_W1bc, ea_W2, ea_b2,
                          wBb, wCb, with_v=False)

    g = _sc_gather(ps_p, pr_p, senders, receivers)
    e_hat = _edge(rel_part, g, re_W2b, re_b2)
    agg = _sc_scatter(e_hat, receivers, zeros_nd)
    v, ps_p, pr_p = _node(obj_part, agg, v, ea_W1bc, ea_W2, ea_b2,
                          wBb, wCb, with_v=True)

    g = _sc_gather(ps_p, pr_p, senders, receivers)
    e_hat = _edge(rel_part, g, re_W2b, re_b2)
    agg = _sc_scatter(e_hat, receivers, zeros_nd)
    out = _node_final(obj_part, agg, v, ea_W1bc, ea_W2, ea_b2,
                      od_W1, od_b1, od_W2, od_b2)
    return out[None]
